# Initial kernel scaffold; baseline (speedup 1.0000x reference)
import jax
import jax.numpy as jnp
from jax.experimental import pallas as pl


def kernel(rays_o, rays_d, rays_d_norm, t_nears, ray_indices, grid_emb, gW1, gb1, gW2, gb2, gW3, gb3, cW1, cb1, cW2, cb2, cW3, cb3, beta):
    raise NotImplementedError("write your pallas kernel here")


# speedup vs baseline: 1.4915x; 1.4915x over previous
import functools

import jax
import jax.numpy as jnp
from jax import lax
from jax.experimental import pallas as pl
from jax.experimental.pallas import tpu as pltpu
from jax.experimental.pallas import tpu_sc as plsc

R = 128
NR = 4096
NS = 262144
B = 512
NBLK = NS // B
NC = 2
NSUB = 16
NW = NC * NSUB
SPW = NS // NW
C = 1024
NCHUNK = SPW // C

_CORNERS = ((0, 0, 0), (1, 0, 0), (0, 1, 0), (1, 1, 0),
            (0, 0, 1), (1, 0, 1), (0, 1, 1), (1, 1, 1))


def _sc_body(ray_tbl, ox_t, oy_t, oz_t, dx_t, dy_t, dz_t, t_hbm, ri_hbm,
             grid_hbm, rayrow_hbm, corners_hbm,
             t_v, ri_v, ox_v, oy_v, oz_v, dx_v, dy_v, dz_v,
             ix0_v, ix1_v, ix2_v, ix3_v, ix4_v, ix5_v, ix6_v, ix7_v,
             rayrow_v, cbuf, sem):
    wid = lax.axis_index("s") * NC + lax.axis_index("c")
    comp_tbls = (ox_t, oy_t, oz_t, dx_t, dy_t, dz_t)
    comp_bufs = (ox_v, oy_v, oz_v, dx_v, dy_v, dz_v)
    idx_bufs = (ix0_v, ix1_v, ix2_v, ix3_v, ix4_v, ix5_v, ix6_v, ix7_v)

    def chunk(ci, carry):
        base = wid * SPW + ci * C
        pltpu.sync_copy(t_hbm.at[pl.ds(base, C)], t_v)
        pltpu.sync_copy(ri_hbm.at[pl.ds(base, C)], ri_v)
        cps = [pltpu.async_copy(ray_tbl.at[ri_v], rayrow_v, sem)]
        for tb, bf in zip(comp_tbls, comp_bufs):
            cps.append(pltpu.async_copy(tb.at[ri_v], bf, sem))
        for cp in cps:
            cp.wait()

        def group(g, carry2):
            sl = pl.ds(g * 16, 16)
            t16 = t_v[sl]
            tm = 0.5 * (t16 + (t16 + 0.01))
            i0 = []
            i1 = []
            for ob, db in ((ox_v, dx_v), (oy_v, dy_v), (oz_v, dz_v)):
                xc = ob[sl] + tm * db[sl]
                u = ((xc + 1.0) * 0.5) * 127.0
                u = jnp.minimum(jnp.maximum(u, 0.0), 126.999999)
                lo = u.astype(jnp.int32)
                i0.append(lo)
                i1.append(jnp.minimum(lo + 1, 127))
            for k, (ax, ay, az) in enumerate(_CORNERS):
                xi = i1[0] if ax else i0[0]
                yi = i1[1] if ay else i0[1]
                zi = i1[2] if az else i0[2]
                idx_bufs[k][sl] = (xi * 128 + yi) * 128 + zi
            return carry2

        lax.fori_loop(0, C // 16, group, 0, unroll=False)
        pltpu.sync_copy(rayrow_v, rayrow_hbm.at[pl.ds(base, C)])
        for k in range(8):
            pltpu.async_copy(grid_hbm.at[idx_bufs[k]], cbuf, sem).wait()
            pltpu.sync_copy(cbuf, corners_hbm.at[k, pl.ds(base, C)])
        return carry

    lax.fori_loop(0, NCHUNK, chunk, 0, unroll=False)


def _sc_gather(ray_tbl, t_nears, ray_indices, grid_flat):
    mesh = plsc.VectorSubcoreMesh(core_axis_name="c", subcore_axis_name="s")
    f = pl.kernel(
        _sc_body,
        out_type=[
            jax.ShapeDtypeStruct((NS, 8), jnp.float32),
            jax.ShapeDtypeStruct((8, NS, 8), jnp.float32),
        ],
        mesh=mesh,
        scratch_types=(
            [pltpu.VMEM((C,), jnp.float32), pltpu.VMEM((C,), jnp.int32)]
            + [pltpu.VMEM((C,), jnp.float32)] * 6
            + [pltpu.VMEM((C,), jnp.int32)] * 8
            + [pltpu.VMEM((C, 8), jnp.float32),
               pltpu.VMEM((C, 8), jnp.float32),
               pltpu.SemaphoreType.DMA]
        ),
        compiler_params=pltpu.CompilerParams(use_tc_tiling_on_sc=False),
    )
    return f(ray_tbl, ray_tbl[:, 0], ray_tbl[:, 1], ray_tbl[:, 2],
             ray_tbl[:, 3], ray_tbl[:, 4], ray_tbl[:, 5], t_nears,
             ray_indices, grid_flat)


def _softplus100(z):
    a = 100.0 * z
    return (jnp.maximum(a, 0.0) + jnp.log(1.0 + jnp.exp(-jnp.abs(a)))) / 100.0


def _sigmoid(z):
    return 1.0 / (1.0 + jnp.exp(-z))


def _tc_body(rayrow_ref, t_ref, rir_ref, ric_ref, corners_ref,
             gW1p_ref, gb1_ref, gW2_ref, gb2_ref, gW3p_ref, gb3p_ref,
             gW1pT_ref, gW2T_ref, g3col_ref, cW1p_ref, cb1_ref, cW2_ref,
             cb2_ref, cW3p_ref, cb3p_ref, tileM_ref, beta_ref,
             sdfg_ref, acc_ref, sm_ref):
    b = pl.program_id(0)

    @pl.when(b == 0)
    def _():
        sm_ref[0] = -1.0
        sm_ref[1] = 0.0

    rayrow = rayrow_ref[...]
    o = rayrow[:, 0:3]
    d = rayrow[:, 3:6]
    t = t_ref[0].reshape(B, 1)
    tf = t + 0.01
    tm = 0.5 * (t + tf)
    delta = tf - t
    x = o + tm * d

    u_pre = ((x + 1.0) * 0.5) * 127.0
    u = jnp.minimum(jnp.maximum(u_pre, 0.0), 126.999999)
    i0f = u.astype(jnp.int32).astype(jnp.float32)
    f = u - i0f
    sj = jnp.where((u_pre > 0.0) & (u_pre < 126.999999), 63.5, 0.0)
    amax = jnp.max(jnp.abs(x), axis=1, keepdims=True)
    maskf = jnp.where(amax <= 1.0, 1.0, 0.0)

    cn = corners_ref[...]
    fx = f[:, 0:1]
    fy = f[:, 1:2]
    fz = f[:, 2:3]
    c00 = cn[0] * (1 - fx) + cn[1] * fx
    c10 = cn[2] * (1 - fx) + cn[3] * fx
    c01 = cn[4] * (1 - fx) + cn[5] * fx
    c11 = cn[6] * (1 - fx) + cn[7] * fx
    c0 = c00 * (1 - fy) + c10 * fy
    c1 = c01 * (1 - fy) + c11 * fy
    emb = (c0 * (1 - fz) + c1 * fz) * maskf
    gx = (((cn[1] - cn[0]) * (1 - fy) + (cn[3] - cn[2]) * fy) * (1 - fz)
          + ((cn[5] - cn[4]) * (1 - fy) + (cn[7] - cn[6]) * fy) * fz)
    gy = (c10 - c00) * (1 - fz) + (c11 - c01) * fz
    gz = c1 - c0

    h0 = jnp.concatenate([x, emb, jnp.zeros((B, 5), jnp.float32)], axis=1)
    z1 = jnp.dot(h0, gW1p_ref[...], preferred_element_type=jnp.float32) + gb1_ref[...]
    a1 = _softplus100(z1)
    z2 = jnp.dot(a1, gW2_ref[...], preferred_element_type=jnp.float32) + gb2_ref[...]
    a2 = _softplus100(z2)
    z3 = jnp.dot(a2, gW3p_ref[...], preferred_element_type=jnp.float32) + gb3p_ref[...]
    sdf = z3[:, 0:1]
    gemb = z3[:, 1:14]

    d2 = _sigmoid(100.0 * z2) * g3col_ref[...]
    d1 = jnp.dot(d2, gW2T_ref[...], preferred_element_type=jnp.float32) * _sigmoid(100.0 * z1)
    dh0 = jnp.dot(d1, gW1pT_ref[...], preferred_element_type=jnp.float32)
    dxd = dh0[:, 0:3]
    demb = dh0[:, 3:11]
    dots = jnp.concatenate(
        [jnp.sum(demb * gx, axis=1, keepdims=True),
         jnp.sum(demb * gy, axis=1, keepdims=True),
         jnp.sum(demb * gz, axis=1, keepdims=True)], axis=1)
    sdfg = dxd + maskf * (sj * dots)
    nrm = jnp.sqrt(jnp.sum(sdfg * sdfg, axis=1, keepdims=True))
    normals = sdfg / jnp.maximum(nrm, 1e-12)

    hc = jnp.concatenate([d, gemb, normals, jnp.zeros((B, 5), jnp.float32)], axis=1)
    zc1 = jnp.dot(hc, cW1p_ref[...], preferred_element_type=jnp.float32) + cb1_ref[...]
    ac1 = jnp.maximum(zc1, 0.0)
    zc2 = jnp.dot(ac1, cW2_ref[...], preferred_element_type=jnp.float32) + cb2_ref[...]
    ac2 = jnp.maximum(zc2, 0.0)
    zc3 = jnp.dot(ac2, cW3p_ref[...], preferred_element_type=jnp.float32) + cb3p_ref[...]
    rgb = _sigmoid(zc3[:, 0:3])

    bc = jnp.maximum(beta_ref[0, 0], 1e-4)
    alpha_v = 1.0 / bc
    sig = 0.5 * alpha_v * (1.0 + jnp.sign(sdf) * (jnp.exp(-jnp.abs(sdf) / bc) - 1.0))
    s_ = sig * delta

    ric = ric_ref[0]
    rir = rir_ref[0]
    ricf = ric.astype(jnp.float32)
    rirf = rir.astype(jnp.float32)
    ii = lax.broadcasted_iota(jnp.int32, (B, B), 0)
    jj = lax.broadcasted_iota(jnp.int32, (B, B), 1)
    Mf = jnp.where((ricf == rirf) & (jj < ii), 1.0, 0.0)
    excl = jnp.dot(Mf, s_, preferred_element_type=jnp.float32)
    carry_ray = sm_ref[0]
    carry_sum = sm_ref[1]
    within = excl + jnp.where(ricf == carry_ray, carry_sum, 0.0)
    T = jnp.exp(-within)
    al = 1.0 - jnp.exp(-s_)
    w = al * T

    lastf = jnp.where(lax.broadcasted_iota(jnp.int32, (B, 1), 0) == B - 1, 1.0, 0.0)
    sm_ref[0] = jnp.sum(ricf * lastf)
    sm_ref[1] = jnp.sum((within + s_) * lastf)

    contrib = jnp.concatenate([w * rgb, w * tm, w * normals, w], axis=1)
    lo_col = lax.bitwise_and(ric, 63)
    hi_row = lax.shift_right_logical(rir, 6)
    iota_c = lax.broadcasted_iota(jnp.int32, (B, B), 1)
    ohLo8 = jnp.where(lax.shift_right_logical(iota_c, 3) == lo_col, 1.0, 0.0)
    ct = jnp.dot(contrib, tileM_ref[...], preferred_element_type=jnp.float32)
    E = ohLo8 * ct
    iota_h = lax.broadcasted_iota(jnp.int32, (64, B), 0)
    ohHiT = jnp.where(iota_h == hi_row, 1.0, 0.0)
    partial = jnp.dot(ohHiT, E, preferred_element_type=jnp.float32)

    sdfg_ref[...] = jnp.concatenate([sdfg, jnp.zeros((B, 5), jnp.float32)], axis=1)

    @pl.when(b == 0)
    def _():
        acc_ref[...] = partial

    @pl.when(b > 0)
    def _():
        acc_ref[...] = acc_ref[...] + partial


def _tc_render(rayrow, t3, rir3, ric3, corners, gW1p, gb1, gW2, gb2, gW3p,
               gb3p, gW1pT, gW2T, g3col, cW1p, cb1, cW2, cb2, cW3p, cb3p,
               tileM, beta2, interpret=False):
    def fullspec(shape, ms=None):
        return pl.BlockSpec(shape, lambda b: tuple(0 for _ in shape),
                            memory_space=ms)

    grid_spec = pltpu.PrefetchScalarGridSpec(
        num_scalar_prefetch=0,
        grid=(NBLK,),
        in_specs=[
            pl.BlockSpec((B, 8), lambda b: (b, 0)),
            pl.BlockSpec((1, 1, B), lambda b: (b, 0, 0)),
            pl.BlockSpec((1, 1, B), lambda b: (b, 0, 0)),
            pl.BlockSpec((1, B, 1), lambda b: (b, 0, 0)),
            pl.BlockSpec((8, B, 8), lambda b: (0, b, 0)),
            fullspec((16, 256)), fullspec((1, 256)),
            fullspec((256, 256)), fullspec((1, 256)),
            fullspec((256, 16)), fullspec((1, 16)),
            fullspec((256, 16)), fullspec((256, 256)), fullspec((1, 256)),
            fullspec((24, 256)), fullspec((1, 256)),
            fullspec((256, 256)), fullspec((1, 256)),
            fullspec((256, 8)), fullspec((1, 8)),
            fullspec((8, B)),
            fullspec((1, 1), pltpu.SMEM),
        ],
        out_specs=[
            pl.BlockSpec((B, 8), lambda b: (b, 0)),
            pl.BlockSpec((64, B), lambda b: (0, 0)),
        ],
        scratch_shapes=[pltpu.SMEM((2,), jnp.float32)],
    )
    return pl.pallas_call(
        _tc_body,
        grid_spec=grid_spec,
        out_shape=[
            jax.ShapeDtypeStruct((NS, 8), jnp.float32),
            jax.ShapeDtypeStruct((64, B), jnp.float32),
        ],
        compiler_params=pltpu.CompilerParams(
            dimension_semantics=("arbitrary",)),
        interpret=interpret,
    )(rayrow, t3, rir3, ric3, corners, gW1p, gb1, gW2, gb2, gW3p, gb3p,
      gW1pT, gW2T, g3col, cW1p, cb1, cW2, cb2, cW3p, cb3p, tileM, beta2)


def _prep_tc_args(rays_o, rays_d, rays_d_norm, t_nears, ray_indices,
                  gW1, gb1, gW2, gb2, gW3, gb3, cW1, cb1, cW2, cb2, cW3,
                  cb3, beta):
    z = jnp.zeros
    f32 = jnp.float32
    gW1p = jnp.concatenate([gW1, z((5, 256), f32)], axis=0)
    gW3p = jnp.concatenate([gW3, z((256, 2), f32)], axis=1)
    gb3p = jnp.concatenate([gb3, z((2,), f32)]).reshape(1, 16)
    gW1pT = gW1p.T
    gW2T = gW2.T
    g3col = gW3[:, 0:1].T
    cW1p = jnp.concatenate([cW1, z((5, 256), f32)], axis=0)
    cW3p = jnp.concatenate([cW3, z((256, 5), f32)], axis=1)
    cb3p = jnp.concatenate([cb3, z((5,), f32)]).reshape(1, 8)
    tileM = (jnp.arange(B, dtype=jnp.int32)[None, :] % 8
             == jnp.arange(8, dtype=jnp.int32)[:, None]).astype(f32)
    t3 = t_nears.reshape(NBLK, 1, B)
    rir3 = ray_indices.reshape(NBLK, 1, B)
    ric3 = ray_indices.reshape(NBLK, B, 1)
    beta2 = beta.reshape(1, 1)
    return (t3, rir3, ric3, gW1p, gb1.reshape(1, 256), gW2,
            gb2.reshape(1, 256), gW3p, gb3p, gW1pT, gW2T, g3col, cW1p,
            cb1.reshape(1, 256), cW2, cb2.reshape(1, 256), cW3p, cb3p,
            tileM, beta2)


def kernel(rays_o, rays_d, rays_d_norm, t_nears, ray_indices, grid_emb,
           gW1, gb1, gW2, gb2, gW3, gb3, cW1, cb1, cW2, cb2, cW3, cb3,
           beta):
    ray_tbl = jnp.concatenate(
        [rays_o, rays_d, jnp.zeros((NR, 2), jnp.float32)], axis=1)
    grid_flat = grid_emb.reshape(R * R * R, 8)
    rayrow, corners = _sc_gather(ray_tbl, t_nears, ray_indices, grid_flat)

    (t3, rir3, ric3, gW1p, gb1r, gW2r, gb2r, gW3p, gb3p, gW1pT, gW2T,
     g3col, cW1p, cb1r, cW2r, cb2r, cW3p, cb3p, tileM, beta2) = \
        _prep_tc_args(rays_o, rays_d, rays_d_norm, t_nears, ray_indices,
                      gW1, gb1, gW2, gb2, gW3, gb3, cW1, cb1, cW2, cb2,
                      cW3, cb3, beta)

    sdfg8, acc64 = _tc_render(rayrow, t3, rir3, ric3, corners, gW1p, gb1r,
                              gW2r, gb2r, gW3p, gb3p, gW1pT, gW2T, g3col,
                              cW1p, cb1r, cW2r, cb2r, cW3p, cb3p, tileM,
                              beta2)
    acc = acc64.reshape(NR, 8)
    rendered_rgb = acc[:, 0:3]
    rendered_depth = acc[:, 3:4] / rays_d_norm
    rendered_normals = acc[:, 4:7]
    accumulated_weights = acc[:, 7:8]
    sdf_grads = sdfg8[:, 0:3]
    return (rendered_rgb, rendered_depth, rendered_normals,
            accumulated_weights, sdf_grads)
